# baseline (speedup 1.0000x reference)
import jax
import jax.numpy as jnp
from jax import lax
from jax.experimental import pallas as pl
from jax.experimental.pallas import tpu as pltpu
from jax.experimental.pallas import tpu_sc as plsc

N = 10000
E = 320000
D = 128
H = 64
C = 16

NC = 2
NS = 16
NW = NC * NS
EPW = E // NW
CK = 80
EPWP = 10240
NCHUNK = EPWP // CK
NBUF = 4
NQUAD = NCHUNK // NBUF
TRASH = 32
RPT = 624
TAIL = N - NS * RPT

_MESH = plsc.VectorSubcoreMesh(core_axis_name="c", subcore_axis_name="s")
_SC_PARAMS = pltpu.CompilerParams(use_tc_tiling_on_sc=False)


def _seg_sum_kernel(width):
    def body(g_hbm, src_hbm, dst_hbm, zeros_hbm, out_hbm,
             src_v, dst_v, rows_a, rows_b, acc_sh, sem_a, sem_b):
        cid = lax.axis_index("c")
        sid = lax.axis_index("s")
        wid = sid * NC + cid

        pltpu.sync_copy(zeros_hbm.at[pl.ds(sid * RPT, RPT)],
                        acc_sh.at[pl.ds(sid * RPT, RPT)])

        @pl.when(sid == 0)
        def _():
            pltpu.sync_copy(zeros_hbm.at[pl.ds(NS * RPT, TAIL)],
                            acc_sh.at[pl.ds(NS * RPT, TAIL)])

        plsc.subcore_barrier()

        pltpu.sync_copy(src_hbm.at[wid], src_v)
        pltpu.sync_copy(dst_hbm.at[wid], dst_v)

        def gather(c, buf, sem):
            return pltpu.async_copy(g_hbm.at[src_v.at[pl.ds(c * CK, CK)]],
                                    buf, sem)

        npair = NCHUNK // 2
        gather(0, rows_a, sem_a)
        gather(1, rows_b, sem_b)

        def pair(i, carry):
            c0 = 2 * i
            pltpu.make_async_copy(g_hbm.at[src_v.at[pl.ds(c0 * CK, CK)]],
                                  rows_a, sem_a).wait()
            pltpu.sync_copy(rows_a, acc_sh.at[dst_v.at[c0]], add=True)

            @pl.when(i < npair - 1)
            def _():
                gather(c0 + 2, rows_a, sem_a)

            pltpu.make_async_copy(g_hbm.at[src_v.at[pl.ds((c0 + 1) * CK, CK)]],
                                  rows_b, sem_b).wait()
            pltpu.sync_copy(rows_b, acc_sh.at[dst_v.at[c0 + 1]], add=True)

            @pl.when(i < npair - 1)
            def _():
                gather(c0 + 3, rows_b, sem_b)

            return carry

        lax.fori_loop(0, npair, pair, 0)
        plsc.subcore_barrier()

        pltpu.sync_copy(acc_sh.at[pl.ds(sid * RPT, RPT)],
                        out_hbm.at[cid].at[pl.ds(sid * RPT, RPT)])

        @pl.when(sid == 0)
        def _():
            pltpu.sync_copy(acc_sh.at[pl.ds(NS * RPT, TAIL)],
                            out_hbm.at[cid].at[pl.ds(NS * RPT, TAIL)])

    return pl.kernel(
        body,
        out_type=jax.ShapeDtypeStruct((NC, N, width), jnp.float32),
        mesh=_MESH,
        scratch_types=[
            pltpu.VMEM((EPWP,), jnp.int32),
            pltpu.VMEM((NCHUNK, CK), jnp.int32),
            pltpu.VMEM((CK, width), jnp.float32),
            pltpu.VMEM((CK, width), jnp.float32),
            pltpu.VMEM_SHARED((N + TRASH, width), jnp.float32),
            pltpu.SemaphoreType.DMA,
            pltpu.SemaphoreType.DMA,
        ],
        compiler_params=_SC_PARAMS,
    )


def _deg_kernel():
    def body(dst_hbm, zeros_hbm, out_hbm, dst_v, ones_v, acc_sh,
             s0, s1, s2, s3):
        ssem = (s0, s1, s2, s3)
        cid = lax.axis_index("c")
        sid = lax.axis_index("s")
        wid = sid * NC + cid

        @pl.when(sid == 0)
        def _():
            pltpu.sync_copy(zeros_hbm, acc_sh.at[pl.ds(0, N)])
        for j in range(CK // 16):
            ones_v[pl.ds(j * 16, 16)] = jnp.full((16,), 1.0, jnp.float32)
        plsc.subcore_barrier()

        pltpu.sync_copy(dst_hbm.at[wid], dst_v)

        def quad(i, carry):
            base = NBUF * i
            for b in range(NBUF):
                @pl.when(i > 0)
                def _(b=b):
                    pltpu.make_async_copy(
                        ones_v, acc_sh.at[dst_v.at[base - NBUF + b]],
                        ssem[b]).wait()
                pltpu.async_copy(ones_v, acc_sh.at[dst_v.at[base + b]],
                                 ssem[b], add=True)
            return carry

        lax.fori_loop(0, NQUAD, quad, 0)
        for b in range(NBUF):
            pltpu.make_async_copy(
                ones_v, acc_sh.at[dst_v.at[NCHUNK - NBUF + b]],
                ssem[b]).wait()
        plsc.subcore_barrier()

        @pl.when(sid == 0)
        def _():
            pltpu.sync_copy(acc_sh.at[pl.ds(0, N)], out_hbm.at[cid])

    return pl.kernel(
        body,
        out_type=jax.ShapeDtypeStruct((NC, N), jnp.float32),
        mesh=_MESH,
        scratch_types=[
            pltpu.VMEM((NCHUNK, CK), jnp.int32),
            pltpu.VMEM((CK,), jnp.float32),
            pltpu.VMEM_SHARED((N + TRASH,), jnp.float32),
        ] + [pltpu.SemaphoreType.DMA] * NBUF,
        compiler_params=_SC_PARAMS,
    )


BN = 1000


def _tc_stage1(x, W1, d0, d1):
    def body(x_ref, w_ref, d0_ref, d1_ref, g1_ref, s1_ref, isr_ref, idg_ref):
        deg = 1.0 + d0_ref[...] + d1_ref[...]
        isr = lax.rsqrt(deg)
        idg = 1.0 / deg
        h = jnp.dot(x_ref[...], w_ref[...], preferred_element_type=jnp.float32)
        g1_ref[...] = h * isr
        s1_ref[...] = h * idg
        isr_ref[...] = isr
        idg_ref[...] = idg

    return pl.pallas_call(
        body,
        grid=(N // BN,),
        in_specs=[
            pl.BlockSpec((BN, D), lambda i: (i, 0)),
            pl.BlockSpec((D, H), lambda i: (0, 0)),
            pl.BlockSpec((BN, 1), lambda i: (i, 0)),
            pl.BlockSpec((BN, 1), lambda i: (i, 0)),
        ],
        out_specs=[
            pl.BlockSpec((BN, H), lambda i: (i, 0)),
            pl.BlockSpec((BN, H), lambda i: (i, 0)),
            pl.BlockSpec((BN, 1), lambda i: (i, 0)),
            pl.BlockSpec((BN, 1), lambda i: (i, 0)),
        ],
        out_shape=[
            jax.ShapeDtypeStruct((N, H), jnp.float32),
            jax.ShapeDtypeStruct((N, H), jnp.float32),
            jax.ShapeDtypeStruct((N, 1), jnp.float32),
            jax.ShapeDtypeStruct((N, 1), jnp.float32),
        ],
    )(x, W1, d0, d1)


def _tc_stage2(a0, a1, s1, isr, idg, b1, W2):
    def body(a0_ref, a1_ref, s1_ref, isr_ref, idg_ref, b1_ref, w_ref,
             g2_ref, s2_ref):
        agg = isr_ref[...] * (a0_ref[...] + a1_ref[...]) + s1_ref[...] + b1_ref[...]
        r = jnp.maximum(agg, 0.0)
        h2 = jnp.dot(r, w_ref[...], preferred_element_type=jnp.float32)
        g2_ref[...] = h2 * isr_ref[...]
        s2_ref[...] = h2 * idg_ref[...]

    return pl.pallas_call(
        body,
        grid=(N // BN,),
        in_specs=[
            pl.BlockSpec((BN, H), lambda i: (i, 0)),
            pl.BlockSpec((BN, H), lambda i: (i, 0)),
            pl.BlockSpec((BN, H), lambda i: (i, 0)),
            pl.BlockSpec((BN, 1), lambda i: (i, 0)),
            pl.BlockSpec((BN, 1), lambda i: (i, 0)),
            pl.BlockSpec((1, H), lambda i: (0, 0)),
            pl.BlockSpec((H, C), lambda i: (0, 0)),
        ],
        out_specs=[
            pl.BlockSpec((BN, C), lambda i: (i, 0)),
            pl.BlockSpec((BN, C), lambda i: (i, 0)),
        ],
        out_shape=[
            jax.ShapeDtypeStruct((N, C), jnp.float32),
            jax.ShapeDtypeStruct((N, C), jnp.float32),
        ],
    )(a0, a1, s1, isr, idg, b1, W2)


def _tc_stage3(a0, a1, s2, isr, b2):
    def body(a0_ref, a1_ref, s2_ref, isr_ref, b2_ref, out_ref):
        out_ref[...] = (isr_ref[...] * (a0_ref[...] + a1_ref[...])
                        + s2_ref[...] + b2_ref[...])

    return pl.pallas_call(
        body,
        grid=(N // BN,),
        in_specs=[
            pl.BlockSpec((BN, C), lambda i: (i, 0)),
            pl.BlockSpec((BN, C), lambda i: (i, 0)),
            pl.BlockSpec((BN, C), lambda i: (i, 0)),
            pl.BlockSpec((BN, 1), lambda i: (i, 0)),
            pl.BlockSpec((1, C), lambda i: (0, 0)),
        ],
        out_specs=pl.BlockSpec((BN, C), lambda i: (i, 0)),
        out_shape=jax.ShapeDtypeStruct((N, C), jnp.float32),
    )(a0, a1, s2, isr, b2)


def kernel(x, edge_index, W1, b1, W2, b2):
    pad = EPWP - EPW
    src = jnp.pad(edge_index[0].reshape(NW, EPW).astype(jnp.int32),
                  ((0, 0), (0, pad)))
    trash = N + jnp.arange(NW, dtype=jnp.int32)[:, None]
    dst = jnp.concatenate(
        [edge_index[1].reshape(NW, EPW).astype(jnp.int32),
         jnp.broadcast_to(trash, (NW, pad))], axis=1)
    dst = dst.reshape(NW, NCHUNK, CK)

    zeros_n = jnp.zeros((N,), jnp.float32)
    zeros_h = jnp.zeros((N, H), jnp.float32)
    zeros_c = jnp.zeros((N, C), jnp.float32)

    degp = _deg_kernel()(dst, zeros_n)
    d0 = degp[0].reshape(N, 1)
    d1 = degp[1].reshape(N, 1)

    g1, s1, isr, idg = _tc_stage1(x, W1, d0, d1)

    agg1 = _seg_sum_kernel(H)(g1, src, dst, zeros_h)
    g2, s2 = _tc_stage2(agg1[0], agg1[1], s1, isr, idg,
                        b1.reshape(1, H), W2)

    agg2 = _seg_sum_kernel(C)(g2, src, dst, zeros_c)
    out = _tc_stage3(agg2[0], agg2[1], s2, isr, b2.reshape(1, C))
    return out


# speedup vs baseline: 1.6390x; 1.6390x over previous
import jax
import jax.numpy as jnp
from jax import lax
from jax.experimental import pallas as pl
from jax.experimental.pallas import tpu as pltpu
from jax.experimental.pallas import tpu_sc as plsc

N = 10000
E = 320000
D = 128
H = 64
C = 16

NC = 2
NS = 16
NW = NC * NS
EPW = E // NW
CK = 80
NCHUNK = EPW // CK
NBUF = 4
NQUAD = (NCHUNK - 1) // NBUF
RPT = 624
TAIL = N - NS * RPT

_MESH = plsc.VectorSubcoreMesh(core_axis_name="c", subcore_axis_name="s")
_SC_PARAMS = pltpu.CompilerParams(use_tc_tiling_on_sc=False)


def _seg_sum_kernel(width):
    def body(g_hbm, src_hbm, dst_hbm, zeros_hbm, out_hbm,
             src_v, dst_v, rows_a, rows_b, acc_sh, sem_a, sem_b):
        cid = lax.axis_index("c")
        sid = lax.axis_index("s")
        wid = sid * NC + cid

        pltpu.sync_copy(zeros_hbm.at[pl.ds(sid * RPT, RPT)],
                        acc_sh.at[pl.ds(sid * RPT, RPT)])

        @pl.when(sid == 0)
        def _():
            pltpu.sync_copy(zeros_hbm.at[pl.ds(NS * RPT, TAIL)],
                            acc_sh.at[pl.ds(NS * RPT, TAIL)])

        plsc.subcore_barrier()

        pltpu.sync_copy(src_hbm.at[wid], src_v)
        pltpu.sync_copy(dst_hbm.at[wid], dst_v)

        def gather(c, buf, sem):
            return pltpu.async_copy(g_hbm.at[src_v.at[pl.ds(c * CK, CK)]],
                                    buf, sem)

        npair = (NCHUNK - 1) // 2
        last = NCHUNK - 1
        gather(0, rows_a, sem_a)
        gather(1, rows_b, sem_b)

        def pair(i, carry):
            c0 = 2 * i
            pltpu.make_async_copy(g_hbm.at[src_v.at[pl.ds(c0 * CK, CK)]],
                                  rows_a, sem_a).wait()
            pltpu.sync_copy(rows_a, acc_sh.at[dst_v.at[c0]], add=True)
            gather(c0 + 2, rows_a, sem_a)

            pltpu.make_async_copy(g_hbm.at[src_v.at[pl.ds((c0 + 1) * CK, CK)]],
                                  rows_b, sem_b).wait()
            pltpu.sync_copy(rows_b, acc_sh.at[dst_v.at[c0 + 1]], add=True)

            @pl.when(i < npair - 1)
            def _():
                gather(c0 + 3, rows_b, sem_b)

            return carry

        lax.fori_loop(0, npair, pair, 0)
        pltpu.make_async_copy(g_hbm.at[src_v.at[pl.ds(last * CK, CK)]],
                              rows_a, sem_a).wait()
        pltpu.sync_copy(rows_a, acc_sh.at[dst_v.at[last]], add=True)
        plsc.subcore_barrier()

        pltpu.sync_copy(acc_sh.at[pl.ds(sid * RPT, RPT)],
                        out_hbm.at[cid].at[pl.ds(sid * RPT, RPT)])

        @pl.when(sid == 0)
        def _():
            pltpu.sync_copy(acc_sh.at[pl.ds(NS * RPT, TAIL)],
                            out_hbm.at[cid].at[pl.ds(NS * RPT, TAIL)])

    return pl.kernel(
        body,
        out_type=jax.ShapeDtypeStruct((NC, N, width), jnp.float32),
        mesh=_MESH,
        scratch_types=[
            pltpu.VMEM((EPW,), jnp.int32),
            pltpu.VMEM((NCHUNK, CK), jnp.int32),
            pltpu.VMEM((CK, width), jnp.float32),
            pltpu.VMEM((CK, width), jnp.float32),
            pltpu.VMEM_SHARED((N, width), jnp.float32),
            pltpu.SemaphoreType.DMA,
            pltpu.SemaphoreType.DMA,
        ],
        compiler_params=_SC_PARAMS,
    )


def _deg_kernel():
    def body(dst_hbm, zeros_hbm, out_hbm, dst_v, ones_v, acc_sh,
             s0, s1, s2, s3):
        ssem = (s0, s1, s2, s3)
        cid = lax.axis_index("c")
        sid = lax.axis_index("s")
        wid = sid * NC + cid

        @pl.when(sid == 0)
        def _():
            pltpu.sync_copy(zeros_hbm, acc_sh.at[pl.ds(0, N)])
        for j in range(CK // 16):
            ones_v[pl.ds(j * 16, 16)] = jnp.full((16,), 1.0, jnp.float32)
        plsc.subcore_barrier()

        pltpu.sync_copy(dst_hbm.at[wid], dst_v)

        def quad(i, carry):
            base = NBUF * i
            for b in range(NBUF):
                @pl.when(i > 0)
                def _(b=b):
                    pltpu.make_async_copy(
                        ones_v, acc_sh.at[dst_v.at[base - NBUF + b]],
                        ssem[b]).wait()
                pltpu.async_copy(ones_v, acc_sh.at[dst_v.at[base + b]],
                                 ssem[b], add=True)
            return carry

        lax.fori_loop(0, NQUAD, quad, 0)
        pltpu.make_async_copy(ones_v, acc_sh.at[dst_v.at[0]], ssem[0]).wait()
        pltpu.async_copy(ones_v, acc_sh.at[dst_v.at[NCHUNK - 1]],
                         ssem[0], add=True)
        for b in range(NBUF):
            pltpu.make_async_copy(ones_v, acc_sh.at[dst_v.at[0]],
                                  ssem[b]).wait()
        plsc.subcore_barrier()

        @pl.when(sid == 0)
        def _():
            pltpu.sync_copy(acc_sh.at[pl.ds(0, N)], out_hbm.at[cid])

    return pl.kernel(
        body,
        out_type=jax.ShapeDtypeStruct((NC, N), jnp.float32),
        mesh=_MESH,
        scratch_types=[
            pltpu.VMEM((NCHUNK, CK), jnp.int32),
            pltpu.VMEM((CK,), jnp.float32),
            pltpu.VMEM_SHARED((N,), jnp.float32),
        ] + [pltpu.SemaphoreType.DMA] * NBUF,
        compiler_params=_SC_PARAMS,
    )


BN = 1000


def _tc_stage1(x, W1, d0, d1):
    def body(x_ref, w_ref, d0_ref, d1_ref, g1_ref, s1_ref, isr_ref, idg_ref):
        deg = 1.0 + d0_ref[...] + d1_ref[...]
        isr = lax.rsqrt(deg)
        idg = 1.0 / deg
        h = jnp.dot(x_ref[...], w_ref[...], preferred_element_type=jnp.float32)
        g1_ref[...] = h * isr
        s1_ref[...] = h * idg
        isr_ref[...] = isr
        idg_ref[...] = idg

    return pl.pallas_call(
        body,
        grid=(N // BN,),
        in_specs=[
            pl.BlockSpec((BN, D), lambda i: (i, 0)),
            pl.BlockSpec((D, H), lambda i: (0, 0)),
            pl.BlockSpec((BN, 1), lambda i: (i, 0)),
            pl.BlockSpec((BN, 1), lambda i: (i, 0)),
        ],
        out_specs=[
            pl.BlockSpec((BN, H), lambda i: (i, 0)),
            pl.BlockSpec((BN, H), lambda i: (i, 0)),
            pl.BlockSpec((BN, 1), lambda i: (i, 0)),
            pl.BlockSpec((BN, 1), lambda i: (i, 0)),
        ],
        out_shape=[
            jax.ShapeDtypeStruct((N, H), jnp.float32),
            jax.ShapeDtypeStruct((N, H), jnp.float32),
            jax.ShapeDtypeStruct((N, 1), jnp.float32),
            jax.ShapeDtypeStruct((N, 1), jnp.float32),
        ],
    )(x, W1, d0, d1)


def _tc_stage2(a0, a1, s1, isr, idg, b1, W2):
    def body(a0_ref, a1_ref, s1_ref, isr_ref, idg_ref, b1_ref, w_ref,
             g2_ref, s2_ref):
        agg = isr_ref[...] * (a0_ref[...] + a1_ref[...]) + s1_ref[...] + b1_ref[...]
        r = jnp.maximum(agg, 0.0)
        h2 = jnp.dot(r, w_ref[...], preferred_element_type=jnp.float32)
        g2_ref[...] = h2 * isr_ref[...]
        s2_ref[...] = h2 * idg_ref[...]

    return pl.pallas_call(
        body,
        grid=(N // BN,),
        in_specs=[
            pl.BlockSpec((BN, H), lambda i: (i, 0)),
            pl.BlockSpec((BN, H), lambda i: (i, 0)),
            pl.BlockSpec((BN, H), lambda i: (i, 0)),
            pl.BlockSpec((BN, 1), lambda i: (i, 0)),
            pl.BlockSpec((BN, 1), lambda i: (i, 0)),
            pl.BlockSpec((1, H), lambda i: (0, 0)),
            pl.BlockSpec((H, C), lambda i: (0, 0)),
        ],
        out_specs=[
            pl.BlockSpec((BN, C), lambda i: (i, 0)),
            pl.BlockSpec((BN, C), lambda i: (i, 0)),
        ],
        out_shape=[
            jax.ShapeDtypeStruct((N, C), jnp.float32),
            jax.ShapeDtypeStruct((N, C), jnp.float32),
        ],
    )(a0, a1, s1, isr, idg, b1, W2)


def _tc_stage3(a0, a1, s2, isr, b2):
    def body(a0_ref, a1_ref, s2_ref, isr_ref, b2_ref, out_ref):
        out_ref[...] = (isr_ref[...] * (a0_ref[...] + a1_ref[...])
                        + s2_ref[...] + b2_ref[...])

    return pl.pallas_call(
        body,
        grid=(N // BN,),
        in_specs=[
            pl.BlockSpec((BN, C), lambda i: (i, 0)),
            pl.BlockSpec((BN, C), lambda i: (i, 0)),
            pl.BlockSpec((BN, C), lambda i: (i, 0)),
            pl.BlockSpec((BN, 1), lambda i: (i, 0)),
            pl.BlockSpec((1, C), lambda i: (0, 0)),
        ],
        out_specs=pl.BlockSpec((BN, C), lambda i: (i, 0)),
        out_shape=jax.ShapeDtypeStruct((N, C), jnp.float32),
    )(a0, a1, s2, isr, b2)


def kernel(x, edge_index, W1, b1, W2, b2):
    src = edge_index[0].reshape(NW, EPW).astype(jnp.int32)
    dst = edge_index[1].reshape(NW, NCHUNK, CK).astype(jnp.int32)

    zeros_n = jnp.zeros((N,), jnp.float32)
    zeros_h = jnp.zeros((N, H), jnp.float32)
    zeros_c = jnp.zeros((N, C), jnp.float32)

    degp = _deg_kernel()(dst, zeros_n)
    d0 = degp[0].reshape(N, 1)
    d1 = degp[1].reshape(N, 1)

    g1, s1, isr, idg = _tc_stage1(x, W1, d0, d1)

    agg1 = _seg_sum_kernel(H)(g1, src, dst, zeros_h)
    g2, s2 = _tc_stage2(agg1[0], agg1[1], s1, isr, idg,
                        b1.reshape(1, H), W2)

    agg2 = _seg_sum_kernel(C)(g2, src, dst, zeros_c)
    out = _tc_stage3(agg2[0], agg2[1], s2, isr, b2.reshape(1, C))
    return out


# speedup vs baseline: 1.8169x; 1.1086x over previous
import jax
import jax.numpy as jnp
from jax import lax
from jax.experimental import pallas as pl
from jax.experimental.pallas import tpu as pltpu
from jax.experimental.pallas import tpu_sc as plsc

N = 10000
E = 320000
D = 128
H = 64
C = 16

NC = 2
NS = 16
NW = NC * NS
EPW = E // NW
CK = 80
NCHUNK = EPW // CK
NBUF = 4
NQUAD = (NCHUNK - 1) // NBUF
RPT = 624
TAIL = N - NS * RPT

_MESH = plsc.VectorSubcoreMesh(core_axis_name="c", subcore_axis_name="s")
_SC_PARAMS = pltpu.CompilerParams(use_tc_tiling_on_sc=False)


def _seg_sum_kernel(width):
    def body(g_hbm, src_hbm, dst_hbm, zeros_hbm, out_hbm,
             src_v, dst_v, rows_a, rows_b, acc_sh, sem_a, sem_b):
        cid = lax.axis_index("c")
        sid = lax.axis_index("s")
        wid = sid * NC + cid

        pltpu.sync_copy(zeros_hbm.at[pl.ds(sid * RPT, RPT)],
                        acc_sh.at[pl.ds(sid * RPT, RPT)])

        @pl.when(sid == 0)
        def _():
            pltpu.sync_copy(zeros_hbm.at[pl.ds(NS * RPT, TAIL)],
                            acc_sh.at[pl.ds(NS * RPT, TAIL)])

        plsc.subcore_barrier()

        pltpu.sync_copy(src_hbm.at[wid], src_v)
        pltpu.sync_copy(dst_hbm.at[wid], dst_v)

        def gather(c, buf, sem):
            return pltpu.async_copy(g_hbm.at[src_v.at[pl.ds(c * CK, CK)]],
                                    buf, sem)

        npair = (NCHUNK - 1) // 2
        last = NCHUNK - 1
        gather(0, rows_a, sem_a)
        gather(1, rows_b, sem_b)

        def pair(i, carry):
            c0 = 2 * i
            pltpu.make_async_copy(g_hbm.at[src_v.at[pl.ds(c0 * CK, CK)]],
                                  rows_a, sem_a).wait()
            pltpu.sync_copy(rows_a, acc_sh.at[dst_v.at[c0]], add=True)
            gather(c0 + 2, rows_a, sem_a)

            pltpu.make_async_copy(g_hbm.at[src_v.at[pl.ds((c0 + 1) * CK, CK)]],
                                  rows_b, sem_b).wait()
            pltpu.sync_copy(rows_b, acc_sh.at[dst_v.at[c0 + 1]], add=True)

            @pl.when(i < npair - 1)
            def _():
                gather(c0 + 3, rows_b, sem_b)

            return carry

        lax.fori_loop(0, npair, pair, 0)
        pltpu.make_async_copy(g_hbm.at[src_v.at[pl.ds(last * CK, CK)]],
                              rows_a, sem_a).wait()
        pltpu.sync_copy(rows_a, acc_sh.at[dst_v.at[last]], add=True)
        plsc.subcore_barrier()

        pltpu.sync_copy(acc_sh.at[pl.ds(sid * RPT, RPT)],
                        out_hbm.at[pl.ds(sid * RPT, RPT),
                                   pl.ds(cid * width, width)])

        @pl.when(sid == 0)
        def _():
            pltpu.sync_copy(acc_sh.at[pl.ds(NS * RPT, TAIL)],
                            out_hbm.at[pl.ds(NS * RPT, TAIL),
                                       pl.ds(cid * width, width)])

    return pl.kernel(
        body,
        out_type=jax.ShapeDtypeStruct((N, 2 * width), jnp.float32),
        mesh=_MESH,
        scratch_types=[
            pltpu.VMEM((EPW,), jnp.int32),
            pltpu.VMEM((NCHUNK, CK), jnp.int32),
            pltpu.VMEM((CK, width), jnp.float32),
            pltpu.VMEM((CK, width), jnp.float32),
            pltpu.VMEM_SHARED((N, width), jnp.float32),
            pltpu.SemaphoreType.DMA,
            pltpu.SemaphoreType.DMA,
        ],
        compiler_params=_SC_PARAMS,
    )


def _deg_kernel():
    def body(dst_hbm, zeros_hbm, out_hbm, dst_v, ones_v, acc_sh,
             s0, s1, s2, s3):
        ssem = (s0, s1, s2, s3)
        cid = lax.axis_index("c")
        sid = lax.axis_index("s")
        wid = sid * NC + cid

        @pl.when(sid == 0)
        def _():
            pltpu.sync_copy(zeros_hbm, acc_sh.at[pl.ds(0, N)])
        for j in range(CK // 16):
            ones_v[pl.ds(j * 16, 16)] = jnp.full((16,), 1.0, jnp.float32)
        plsc.subcore_barrier()

        pltpu.sync_copy(dst_hbm.at[wid], dst_v)

        def quad(i, carry):
            base = NBUF * i
            for b in range(NBUF):
                @pl.when(i > 0)
                def _(b=b):
                    pltpu.make_async_copy(
                        ones_v, acc_sh.at[dst_v.at[base - NBUF + b]],
                        ssem[b]).wait()
                pltpu.async_copy(ones_v, acc_sh.at[dst_v.at[base + b]],
                                 ssem[b], add=True)
            return carry

        lax.fori_loop(0, NQUAD, quad, 0)
        pltpu.make_async_copy(ones_v, acc_sh.at[dst_v.at[0]], ssem[0]).wait()
        pltpu.async_copy(ones_v, acc_sh.at[dst_v.at[NCHUNK - 1]],
                         ssem[0], add=True)
        for b in range(NBUF):
            pltpu.make_async_copy(ones_v, acc_sh.at[dst_v.at[0]],
                                  ssem[b]).wait()
        plsc.subcore_barrier()

        @pl.when(sid == 0)
        def _():
            pltpu.sync_copy(acc_sh.at[pl.ds(0, N)], out_hbm.at[cid])

    return pl.kernel(
        body,
        out_type=jax.ShapeDtypeStruct((NC, N), jnp.float32),
        mesh=_MESH,
        scratch_types=[
            pltpu.VMEM((NCHUNK, CK), jnp.int32),
            pltpu.VMEM((CK,), jnp.float32),
            pltpu.VMEM_SHARED((N,), jnp.float32),
        ] + [pltpu.SemaphoreType.DMA] * NBUF,
        compiler_params=_SC_PARAMS,
    )


BN = 1000


def _tc_stage1(x, W1, d0, d1):
    def body(x_ref, w_ref, d0_ref, d1_ref, g1_ref, p1_ref):
        deg = 1.0 + d0_ref[...] + d1_ref[...]
        isr = lax.rsqrt(deg)
        idg = 1.0 / deg
        h = jnp.dot(x_ref[...], w_ref[...], preferred_element_type=jnp.float32)
        g1_ref[...] = h * isr
        p1_ref[:, :H] = h * idg
        p1_ref[:, H:H + 1] = isr
        p1_ref[:, H + 1:H + 2] = idg

    return pl.pallas_call(
        body,
        grid=(N // BN,),
        in_specs=[
            pl.BlockSpec((BN, D), lambda i: (i, 0)),
            pl.BlockSpec((D, H), lambda i: (0, 0)),
            pl.BlockSpec((BN, 1), lambda i: (i, 0)),
            pl.BlockSpec((BN, 1), lambda i: (i, 0)),
        ],
        out_specs=[
            pl.BlockSpec((BN, H), lambda i: (i, 0)),
            pl.BlockSpec((BN, 128), lambda i: (i, 0)),
        ],
        out_shape=[
            jax.ShapeDtypeStruct((N, H), jnp.float32),
            jax.ShapeDtypeStruct((N, 128), jnp.float32),
        ],
    )(x, W1, d0, d1)


def _tc_stage2(agg, p1, b1, W2):
    def body(agg_ref, p1_ref, b1_ref, w_ref, g2_ref, p2_ref):
        a = agg_ref[:, :H] + agg_ref[:, H:]
        isr = p1_ref[:, H:H + 1]
        idg = p1_ref[:, H + 1:H + 2]
        r = jnp.maximum(isr * a + p1_ref[:, :H] + b1_ref[...], 0.0)
        h2 = jnp.dot(r, w_ref[...], preferred_element_type=jnp.float32)
        g2_ref[...] = h2 * isr
        p2_ref[:, :C] = h2 * idg
        p2_ref[:, C:C + 1] = isr

    return pl.pallas_call(
        body,
        grid=(N // BN,),
        in_specs=[
            pl.BlockSpec((BN, 2 * H), lambda i: (i, 0)),
            pl.BlockSpec((BN, 128), lambda i: (i, 0)),
            pl.BlockSpec((1, H), lambda i: (0, 0)),
            pl.BlockSpec((H, C), lambda i: (0, 0)),
        ],
        out_specs=[
            pl.BlockSpec((BN, C), lambda i: (i, 0)),
            pl.BlockSpec((BN, 128), lambda i: (i, 0)),
        ],
        out_shape=[
            jax.ShapeDtypeStruct((N, C), jnp.float32),
            jax.ShapeDtypeStruct((N, 128), jnp.float32),
        ],
    )(agg, p1, b1, W2)


def _tc_stage3(agg, p2, b2):
    def body(agg_ref, p2_ref, b2_ref, out_ref):
        a = agg_ref[:, :C] + agg_ref[:, C:]
        isr = p2_ref[:, C:C + 1]
        out_ref[...] = isr * a + p2_ref[:, :C] + b2_ref[...]

    return pl.pallas_call(
        body,
        grid=(N // BN,),
        in_specs=[
            pl.BlockSpec((BN, 2 * C), lambda i: (i, 0)),
            pl.BlockSpec((BN, 128), lambda i: (i, 0)),
            pl.BlockSpec((1, C), lambda i: (0, 0)),
        ],
        out_specs=pl.BlockSpec((BN, C), lambda i: (i, 0)),
        out_shape=jax.ShapeDtypeStruct((N, C), jnp.float32),
    )(agg, p2, b2)


def kernel(x, edge_index, W1, b1, W2, b2):
    src = edge_index[0].reshape(NW, EPW).astype(jnp.int32)
    dst = edge_index[1].reshape(NW, NCHUNK, CK).astype(jnp.int32)

    zeros_n = jnp.zeros((N,), jnp.float32)
    zeros_h = jnp.zeros((N, H), jnp.float32)
    zeros_c = jnp.zeros((N, C), jnp.float32)

    degp = _deg_kernel()(dst, zeros_n)
    d0 = degp[0].reshape(N, 1)
    d1 = degp[1].reshape(N, 1)

    g1, p1 = _tc_stage1(x, W1, d0, d1)

    agg1 = _seg_sum_kernel(H)(g1, src, dst, zeros_h)
    g2, p2 = _tc_stage2(agg1, p1, b1.reshape(1, H), W2)

    agg2 = _seg_sum_kernel(C)(g2, src, dst, zeros_c)
    out = _tc_stage3(agg2, p2, b2.reshape(1, C))
    return out


# speedup vs baseline: 1.8388x; 1.0121x over previous
import jax
import jax.numpy as jnp
from jax import lax
from jax.experimental import pallas as pl
from jax.experimental.pallas import tpu as pltpu
from jax.experimental.pallas import tpu_sc as plsc

N = 10000
E = 320000
D = 128
H = 64
C = 16

NC = 2
NS = 16
NW = NC * NS
EPW = E // NW
CK = 80
NCHUNK = EPW // CK
NBUF = 4
NQUAD = (NCHUNK - 1) // NBUF
RPT = 624
TAIL = N - NS * RPT

_MESH = plsc.VectorSubcoreMesh(core_axis_name="c", subcore_axis_name="s")
_SC_PARAMS = pltpu.CompilerParams(use_tc_tiling_on_sc=False)


def _seg_sum_kernel(width):
    def body(g_hbm, src_hbm, dst_hbm, zeros_hbm, out_hbm,
             src_v, dst_v, rows_a, rows_b, acc_sh, sem_a, sem_b):
        cid = lax.axis_index("c")
        sid = lax.axis_index("s")
        wid = sid * NC + cid

        pltpu.sync_copy(zeros_hbm.at[pl.ds(sid * RPT, RPT)],
                        acc_sh.at[pl.ds(sid * RPT, RPT)])

        @pl.when(sid == 0)
        def _():
            pltpu.sync_copy(zeros_hbm.at[pl.ds(NS * RPT, TAIL)],
                            acc_sh.at[pl.ds(NS * RPT, TAIL)])

        plsc.subcore_barrier()

        pltpu.sync_copy(src_hbm.at[wid], src_v)
        pltpu.sync_copy(dst_hbm.at[wid], dst_v)

        def gather(c, buf, sem):
            return pltpu.async_copy(g_hbm.at[src_v.at[pl.ds(c * CK, CK)]],
                                    buf, sem)

        npair = (NCHUNK - 1) // 2
        last = NCHUNK - 1
        gather(0, rows_a, sem_a)
        gather(1, rows_b, sem_b)

        def pair(i, carry):
            c0 = 2 * i
            pltpu.make_async_copy(g_hbm.at[src_v.at[pl.ds(c0 * CK, CK)]],
                                  rows_a, sem_a).wait()
            pltpu.sync_copy(rows_a, acc_sh.at[dst_v.at[c0]], add=True)
            gather(c0 + 2, rows_a, sem_a)

            pltpu.make_async_copy(g_hbm.at[src_v.at[pl.ds((c0 + 1) * CK, CK)]],
                                  rows_b, sem_b).wait()
            pltpu.sync_copy(rows_b, acc_sh.at[dst_v.at[c0 + 1]], add=True)

            @pl.when(i < npair - 1)
            def _():
                gather(c0 + 3, rows_b, sem_b)

            return carry

        lax.fori_loop(0, npair, pair, 0)
        pltpu.make_async_copy(g_hbm.at[src_v.at[pl.ds(last * CK, CK)]],
                              rows_a, sem_a).wait()
        pltpu.sync_copy(rows_a, acc_sh.at[dst_v.at[last]], add=True)
        plsc.subcore_barrier()

        pltpu.sync_copy(acc_sh.at[pl.ds(sid * RPT, RPT)],
                        out_hbm.at[pl.ds(sid * RPT, RPT),
                                   pl.ds(cid * width, width)])

        @pl.when(sid == 0)
        def _():
            pltpu.sync_copy(acc_sh.at[pl.ds(NS * RPT, TAIL)],
                            out_hbm.at[pl.ds(NS * RPT, TAIL),
                                       pl.ds(cid * width, width)])

    return pl.kernel(
        body,
        out_type=jax.ShapeDtypeStruct((N, 2 * width), jnp.float32),
        mesh=_MESH,
        scratch_types=[
            pltpu.VMEM((EPW,), jnp.int32),
            pltpu.VMEM((NCHUNK, CK), jnp.int32),
            pltpu.VMEM((CK, width), jnp.float32),
            pltpu.VMEM((CK, width), jnp.float32),
            pltpu.VMEM_SHARED((N, width), jnp.float32),
            pltpu.SemaphoreType.DMA,
            pltpu.SemaphoreType.DMA,
        ],
        compiler_params=_SC_PARAMS,
    )


def _deg_kernel():
    def body(dst_hbm, zeros_hbm, out_hbm, dst_v, ones_v, acc_sh,
             s0, s1, s2, s3):
        ssem = (s0, s1, s2, s3)
        cid = lax.axis_index("c")
        sid = lax.axis_index("s")
        wid = sid * NC + cid

        @pl.when(sid == 0)
        def _():
            pltpu.sync_copy(zeros_hbm, acc_sh.at[pl.ds(0, N)])
        for j in range(CK // 16):
            ones_v[pl.ds(j * 16, 16)] = jnp.full((16,), 1.0, jnp.float32)
        plsc.subcore_barrier()

        pltpu.sync_copy(dst_hbm.at[wid], dst_v)

        def quad(i, carry):
            base = NBUF * i
            for b in range(NBUF):
                @pl.when(i > 0)
                def _(b=b):
                    pltpu.make_async_copy(
                        ones_v, acc_sh.at[dst_v.at[base - NBUF + b]],
                        ssem[b]).wait()
                pltpu.async_copy(ones_v, acc_sh.at[dst_v.at[base + b]],
                                 ssem[b], add=True)
            return carry

        lax.fori_loop(0, NQUAD, quad, 0)
        pltpu.make_async_copy(ones_v, acc_sh.at[dst_v.at[0]], ssem[0]).wait()
        pltpu.async_copy(ones_v, acc_sh.at[dst_v.at[NCHUNK - 1]],
                         ssem[0], add=True)
        for b in range(NBUF):
            pltpu.make_async_copy(ones_v, acc_sh.at[dst_v.at[0]],
                                  ssem[b]).wait()
        plsc.subcore_barrier()

        @pl.when(sid == 0)
        def _():
            pltpu.sync_copy(acc_sh.at[pl.ds(0, N)], out_hbm.at[cid])

    return pl.kernel(
        body,
        out_type=jax.ShapeDtypeStruct((NC, N), jnp.float32),
        mesh=_MESH,
        scratch_types=[
            pltpu.VMEM((NCHUNK, CK), jnp.int32),
            pltpu.VMEM((CK,), jnp.float32),
            pltpu.VMEM_SHARED((N,), jnp.float32),
        ] + [pltpu.SemaphoreType.DMA] * NBUF,
        compiler_params=_SC_PARAMS,
    )


BN = 2000


def _tc_stage1(x, W1, d0, d1):
    def body(x_ref, w_ref, d0_ref, d1_ref, g1_ref, p1_ref):
        deg = 1.0 + d0_ref[...] + d1_ref[...]
        isr = lax.rsqrt(deg)
        idg = 1.0 / deg
        h = jnp.dot(x_ref[...], w_ref[...], preferred_element_type=jnp.float32)
        g1_ref[...] = h * isr
        p1_ref[:, :H] = h * idg
        p1_ref[:, H:H + 1] = isr
        p1_ref[:, H + 1:H + 2] = idg

    return pl.pallas_call(
        body,
        grid=(N // BN,),
        in_specs=[
            pl.BlockSpec((BN, D), lambda i: (i, 0)),
            pl.BlockSpec((D, H), lambda i: (0, 0)),
            pl.BlockSpec((BN, 1), lambda i: (i, 0)),
            pl.BlockSpec((BN, 1), lambda i: (i, 0)),
        ],
        out_specs=[
            pl.BlockSpec((BN, H), lambda i: (i, 0)),
            pl.BlockSpec((BN, 128), lambda i: (i, 0)),
        ],
        out_shape=[
            jax.ShapeDtypeStruct((N, H), jnp.float32),
            jax.ShapeDtypeStruct((N, 128), jnp.float32),
        ],
    )(x, W1, d0, d1)


def _tc_stage2(agg, p1, b1, W2):
    def body(agg_ref, p1_ref, b1_ref, w_ref, g2_ref, p2_ref):
        a = agg_ref[:, :H] + agg_ref[:, H:]
        isr = p1_ref[:, H:H + 1]
        idg = p1_ref[:, H + 1:H + 2]
        r = jnp.maximum(isr * a + p1_ref[:, :H] + b1_ref[...], 0.0)
        h2 = jnp.dot(r, w_ref[...], preferred_element_type=jnp.float32)
        g2_ref[...] = h2 * isr
        p2_ref[:, :C] = h2 * idg
        p2_ref[:, C:C + 1] = isr

    return pl.pallas_call(
        body,
        grid=(N // BN,),
        in_specs=[
            pl.BlockSpec((BN, 2 * H), lambda i: (i, 0)),
            pl.BlockSpec((BN, 128), lambda i: (i, 0)),
            pl.BlockSpec((1, H), lambda i: (0, 0)),
            pl.BlockSpec((H, C), lambda i: (0, 0)),
        ],
        out_specs=[
            pl.BlockSpec((BN, C), lambda i: (i, 0)),
            pl.BlockSpec((BN, 128), lambda i: (i, 0)),
        ],
        out_shape=[
            jax.ShapeDtypeStruct((N, C), jnp.float32),
            jax.ShapeDtypeStruct((N, 128), jnp.float32),
        ],
    )(agg, p1, b1, W2)


def _tc_stage3(agg, p2, b2):
    def body(agg_ref, p2_ref, b2_ref, out_ref):
        a = agg_ref[:, :C] + agg_ref[:, C:]
        isr = p2_ref[:, C:C + 1]
        out_ref[...] = isr * a + p2_ref[:, :C] + b2_ref[...]

    return pl.pallas_call(
        body,
        grid=(N // BN,),
        in_specs=[
            pl.BlockSpec((BN, 2 * C), lambda i: (i, 0)),
            pl.BlockSpec((BN, 128), lambda i: (i, 0)),
            pl.BlockSpec((1, C), lambda i: (0, 0)),
        ],
        out_specs=pl.BlockSpec((BN, C), lambda i: (i, 0)),
        out_shape=jax.ShapeDtypeStruct((N, C), jnp.float32),
    )(agg, p2, b2)


def kernel(x, edge_index, W1, b1, W2, b2):
    er = edge_index.reshape(2 * E)
    src = er[:E].reshape(NW, EPW)
    dst = er[E:].reshape(NW, NCHUNK, CK)

    zeros_n = jnp.zeros((N,), jnp.float32)
    zeros_h = jnp.zeros((N, H), jnp.float32)
    zeros_c = jnp.zeros((N, C), jnp.float32)

    degp = _deg_kernel()(dst, zeros_n)
    d0 = degp[0].reshape(N, 1)
    d1 = degp[1].reshape(N, 1)

    g1, p1 = _tc_stage1(x, W1, d0, d1)

    agg1 = _seg_sum_kernel(H)(g1, src, dst, zeros_h)
    g2, p2 = _tc_stage2(agg1, p1, b1.reshape(1, H), W2)

    agg2 = _seg_sum_kernel(C)(g2, src, dst, zeros_c)
    out = _tc_stage3(agg2, p2, b2.reshape(1, C))
    return out


# speedup vs baseline: 1.9544x; 1.0629x over previous
import jax
import jax.numpy as jnp
from jax import lax
from jax.experimental import pallas as pl
from jax.experimental.pallas import tpu as pltpu
from jax.experimental.pallas import tpu_sc as plsc

N = 10000
E = 320000
D = 128
H = 64
C = 16

NC = 2
NS = 16
NW = NC * NS
EPW = E // NW
CK = 80
NCHUNK = EPW // CK
NBUF = 4
NQUAD = (NCHUNK - 1) // NBUF
RPT = 624
TAIL = N - NS * RPT

_MESH = plsc.VectorSubcoreMesh(core_axis_name="c", subcore_axis_name="s")
_SC_PARAMS = pltpu.CompilerParams(use_tc_tiling_on_sc=False)


def _seg_sum_kernel(width):
    def body(g_hbm, ei_hbm, zeros_hbm, out_hbm,
             src_v, dst_v, rows_a, rows_b, acc_sh, sem_a, sem_b):
        cid = lax.axis_index("c")
        sid = lax.axis_index("s")
        wid = sid * NC + cid

        pltpu.sync_copy(zeros_hbm.at[pl.ds(sid * RPT, RPT)],
                        acc_sh.at[pl.ds(sid * RPT, RPT)])

        @pl.when(sid == 0)
        def _():
            pltpu.sync_copy(zeros_hbm.at[pl.ds(NS * RPT, TAIL)],
                            acc_sh.at[pl.ds(NS * RPT, TAIL)])

        plsc.subcore_barrier()

        pltpu.sync_copy(ei_hbm.at[0].at[pl.ds(wid * EPW, EPW)], src_v)
        pltpu.sync_copy(ei_hbm.at[1].at[pl.ds(wid * EPW, EPW)], dst_v)

        def gather(c, buf, sem):
            return pltpu.async_copy(g_hbm.at[src_v.at[pl.ds(c * CK, CK)]],
                                    buf, sem)

        npair = (NCHUNK - 1) // 2
        last = NCHUNK - 1
        gather(0, rows_a, sem_a)
        gather(1, rows_b, sem_b)

        def pair(i, carry):
            c0 = 2 * i
            pltpu.make_async_copy(g_hbm.at[src_v.at[pl.ds(c0 * CK, CK)]],
                                  rows_a, sem_a).wait()
            pltpu.sync_copy(rows_a, acc_sh.at[dst_v.at[pl.ds(c0 * CK, CK)]], add=True)
            gather(c0 + 2, rows_a, sem_a)

            pltpu.make_async_copy(g_hbm.at[src_v.at[pl.ds((c0 + 1) * CK, CK)]],
                                  rows_b, sem_b).wait()
            pltpu.sync_copy(rows_b, acc_sh.at[dst_v.at[pl.ds((c0 + 1) * CK, CK)]], add=True)

            @pl.when(i < npair - 1)
            def _():
                gather(c0 + 3, rows_b, sem_b)

            return carry

        lax.fori_loop(0, npair, pair, 0)
        pltpu.make_async_copy(g_hbm.at[src_v.at[pl.ds(last * CK, CK)]],
                              rows_a, sem_a).wait()
        pltpu.sync_copy(rows_a, acc_sh.at[dst_v.at[pl.ds(last * CK, CK)]], add=True)
        plsc.subcore_barrier()

        pltpu.sync_copy(acc_sh.at[pl.ds(sid * RPT, RPT)],
                        out_hbm.at[pl.ds(sid * RPT, RPT),
                                   pl.ds(cid * width, width)])

        @pl.when(sid == 0)
        def _():
            pltpu.sync_copy(acc_sh.at[pl.ds(NS * RPT, TAIL)],
                            out_hbm.at[pl.ds(NS * RPT, TAIL),
                                       pl.ds(cid * width, width)])

    return pl.kernel(
        body,
        out_type=jax.ShapeDtypeStruct((N, 2 * width), jnp.float32),
        mesh=_MESH,
        scratch_types=[
            pltpu.VMEM((EPW,), jnp.int32),
            pltpu.VMEM((EPW,), jnp.int32),
            pltpu.VMEM((CK, width), jnp.float32),
            pltpu.VMEM((CK, width), jnp.float32),
            pltpu.VMEM_SHARED((N, width), jnp.float32),
            pltpu.SemaphoreType.DMA,
            pltpu.SemaphoreType.DMA,
        ],
        compiler_params=_SC_PARAMS,
    )


def _deg_kernel():
    def body(ei_hbm, zeros_hbm, out_hbm, dst_v, ones_v, acc_sh,
             s0, s1, s2, s3):
        ssem = (s0, s1, s2, s3)
        cid = lax.axis_index("c")
        sid = lax.axis_index("s")
        wid = sid * NC + cid

        @pl.when(sid == 0)
        def _():
            pltpu.sync_copy(zeros_hbm, acc_sh.at[pl.ds(0, N)])
        for j in range(CK // 16):
            ones_v[pl.ds(j * 16, 16)] = jnp.full((16,), 1.0, jnp.float32)
        plsc.subcore_barrier()

        pltpu.sync_copy(ei_hbm.at[1].at[pl.ds(wid * EPW, EPW)], dst_v)

        def quad(i, carry):
            base = NBUF * i
            for b in range(NBUF):
                @pl.when(i > 0)
                def _(b=b):
                    pltpu.make_async_copy(
                        ones_v, acc_sh.at[dst_v.at[pl.ds((base - NBUF + b) * CK, CK)]],
                        ssem[b]).wait()
                pltpu.async_copy(ones_v, acc_sh.at[dst_v.at[pl.ds((base + b) * CK, CK)]],
                                 ssem[b], add=True)
            return carry

        lax.fori_loop(0, NQUAD, quad, 0)
        pltpu.make_async_copy(ones_v, acc_sh.at[dst_v.at[pl.ds(0, CK)]], ssem[0]).wait()
        pltpu.async_copy(ones_v, acc_sh.at[dst_v.at[pl.ds((NCHUNK - 1) * CK, CK)]],
                         ssem[0], add=True)
        for b in range(NBUF):
            pltpu.make_async_copy(ones_v, acc_sh.at[dst_v.at[pl.ds(0, CK)]],
                                  ssem[b]).wait()
        plsc.subcore_barrier()

        @pl.when(sid == 0)
        def _():
            pltpu.sync_copy(acc_sh.at[pl.ds(0, N)], out_hbm.at[cid])

    return pl.kernel(
        body,
        out_type=jax.ShapeDtypeStruct((NC, N), jnp.float32),
        mesh=_MESH,
        scratch_types=[
            pltpu.VMEM((EPW,), jnp.int32),
            pltpu.VMEM((CK,), jnp.float32),
            pltpu.VMEM_SHARED((N,), jnp.float32),
        ] + [pltpu.SemaphoreType.DMA] * NBUF,
        compiler_params=_SC_PARAMS,
    )


BN = 2000


def _tc_stage1(x, W1, d0, d1):
    def body(x_ref, w_ref, d0_ref, d1_ref, g1_ref, p1_ref):
        deg = 1.0 + d0_ref[...] + d1_ref[...]
        isr = lax.rsqrt(deg)
        idg = 1.0 / deg
        h = jnp.dot(x_ref[...], w_ref[...], preferred_element_type=jnp.float32)
        g1_ref[...] = h * isr
        p1_ref[:, :H] = h * idg
        p1_ref[:, H:H + 1] = isr
        p1_ref[:, H + 1:H + 2] = idg

    return pl.pallas_call(
        body,
        grid=(N // BN,),
        in_specs=[
            pl.BlockSpec((BN, D), lambda i: (i, 0)),
            pl.BlockSpec((D, H), lambda i: (0, 0)),
            pl.BlockSpec((BN, 1), lambda i: (i, 0)),
            pl.BlockSpec((BN, 1), lambda i: (i, 0)),
        ],
        out_specs=[
            pl.BlockSpec((BN, H), lambda i: (i, 0)),
            pl.BlockSpec((BN, 128), lambda i: (i, 0)),
        ],
        out_shape=[
            jax.ShapeDtypeStruct((N, H), jnp.float32),
            jax.ShapeDtypeStruct((N, 128), jnp.float32),
        ],
    )(x, W1, d0, d1)


def _tc_stage2(agg, p1, b1, W2):
    def body(agg_ref, p1_ref, b1_ref, w_ref, g2_ref, p2_ref):
        a = agg_ref[:, :H] + agg_ref[:, H:]
        isr = p1_ref[:, H:H + 1]
        idg = p1_ref[:, H + 1:H + 2]
        r = jnp.maximum(isr * a + p1_ref[:, :H] + b1_ref[...], 0.0)
        h2 = jnp.dot(r, w_ref[...], preferred_element_type=jnp.float32)
        g2_ref[...] = h2 * isr
        p2_ref[:, :C] = h2 * idg
        p2_ref[:, C:C + 1] = isr

    return pl.pallas_call(
        body,
        grid=(N // BN,),
        in_specs=[
            pl.BlockSpec((BN, 2 * H), lambda i: (i, 0)),
            pl.BlockSpec((BN, 128), lambda i: (i, 0)),
            pl.BlockSpec((1, H), lambda i: (0, 0)),
            pl.BlockSpec((H, C), lambda i: (0, 0)),
        ],
        out_specs=[
            pl.BlockSpec((BN, C), lambda i: (i, 0)),
            pl.BlockSpec((BN, 128), lambda i: (i, 0)),
        ],
        out_shape=[
            jax.ShapeDtypeStruct((N, C), jnp.float32),
            jax.ShapeDtypeStruct((N, 128), jnp.float32),
        ],
    )(agg, p1, b1, W2)


def _tc_stage3(agg, p2, b2):
    def body(agg_ref, p2_ref, b2_ref, out_ref):
        a = agg_ref[:, :C] + agg_ref[:, C:]
        isr = p2_ref[:, C:C + 1]
        out_ref[...] = isr * a + p2_ref[:, :C] + b2_ref[...]

    return pl.pallas_call(
        body,
        grid=(N // BN,),
        in_specs=[
            pl.BlockSpec((BN, 2 * C), lambda i: (i, 0)),
            pl.BlockSpec((BN, 128), lambda i: (i, 0)),
            pl.BlockSpec((1, C), lambda i: (0, 0)),
        ],
        out_specs=pl.BlockSpec((BN, C), lambda i: (i, 0)),
        out_shape=jax.ShapeDtypeStruct((N, C), jnp.float32),
    )(agg, p2, b2)


def kernel(x, edge_index, W1, b1, W2, b2):
    zeros_n = jnp.zeros((N,), jnp.float32)
    zeros_h = jnp.zeros((N, H), jnp.float32)
    zeros_c = jnp.zeros((N, C), jnp.float32)

    ei = edge_index.astype(jnp.int32)
    degp = _deg_kernel()(ei, zeros_n)
    d0 = degp[0].reshape(N, 1)
    d1 = degp[1].reshape(N, 1)

    g1, p1 = _tc_stage1(x, W1, d0, d1)

    agg1 = _seg_sum_kernel(H)(g1, ei, zeros_h)
    g2, p2 = _tc_stage2(agg1, p1, b1.reshape(1, H), W2)

    agg2 = _seg_sum_kernel(C)(g2, ei, zeros_c)
    out = _tc_stage3(agg2, p2, b2.reshape(1, C))
    return out


# speedup vs baseline: 2.1603x; 1.1053x over previous
import jax
import jax.numpy as jnp
from jax import lax
from jax.experimental import pallas as pl
from jax.experimental.pallas import tpu as pltpu
from jax.experimental.pallas import tpu_sc as plsc

N = 10000
E = 320000
D = 128
H = 64
C = 16

NC = 2
NS = 16
NW = NC * NS
EPW = E // NW
CK = 112
NFULL = EPW // CK
CKT = EPW - NFULL * CK
TOFF = NFULL * CK
NBUF = 4
NQUAD = (NFULL - 1) // NBUF
RPT = 624
TAIL = N - NS * RPT

_MESH = plsc.VectorSubcoreMesh(core_axis_name="c", subcore_axis_name="s")
_SC_PARAMS = pltpu.CompilerParams(use_tc_tiling_on_sc=False)


def _seg_sum_kernel(width):
    def body(g_hbm, ei_hbm, zeros_hbm, out_hbm,
             src_v, dst_v, rows_a, rows_b, rows_t, acc_sh,
             sem_a, sem_b, sem_t):
        cid = lax.axis_index("c")
        sid = lax.axis_index("s")
        wid = sid * NC + cid

        pltpu.sync_copy(zeros_hbm.at[pl.ds(sid * RPT, RPT)],
                        acc_sh.at[pl.ds(sid * RPT, RPT)])

        @pl.when(sid == 0)
        def _():
            pltpu.sync_copy(zeros_hbm.at[pl.ds(NS * RPT, TAIL)],
                            acc_sh.at[pl.ds(NS * RPT, TAIL)])

        plsc.subcore_barrier()

        pltpu.sync_copy(ei_hbm.at[0].at[pl.ds(wid * EPW, EPW)], src_v)
        pltpu.sync_copy(ei_hbm.at[1].at[pl.ds(wid * EPW, EPW)], dst_v)

        def gather(c, buf, sem):
            return pltpu.async_copy(g_hbm.at[src_v.at[pl.ds(c * CK, CK)]],
                                    buf, sem)

        npair = NFULL // 2
        gather(0, rows_a, sem_a)
        gather(1, rows_b, sem_b)
        pltpu.async_copy(g_hbm.at[src_v.at[pl.ds(TOFF, CKT)]], rows_t, sem_t)

        def pair(i, carry):
            c0 = 2 * i
            pltpu.make_async_copy(g_hbm.at[src_v.at[pl.ds(c0 * CK, CK)]],
                                  rows_a, sem_a).wait()
            pltpu.sync_copy(rows_a, acc_sh.at[dst_v.at[pl.ds(c0 * CK, CK)]],
                            add=True)
            gather(c0 + 2, rows_a, sem_a)

            pltpu.make_async_copy(g_hbm.at[src_v.at[pl.ds((c0 + 1) * CK, CK)]],
                                  rows_b, sem_b).wait()
            pltpu.sync_copy(rows_b, acc_sh.at[dst_v.at[pl.ds((c0 + 1) * CK, CK)]],
                            add=True)

            @pl.when(i < npair - 1)
            def _():
                gather(c0 + 3, rows_b, sem_b)

            return carry

        lax.fori_loop(0, npair, pair, 0)
        pltpu.make_async_copy(g_hbm.at[src_v.at[pl.ds(NFULL * CK - CK, CK)]],
                              rows_a, sem_a).wait()
        pltpu.sync_copy(rows_a,
                        acc_sh.at[dst_v.at[pl.ds((NFULL - 1) * CK, CK)]],
                        add=True)
        pltpu.make_async_copy(g_hbm.at[src_v.at[pl.ds(TOFF, CKT)]],
                              rows_t, sem_t).wait()
        pltpu.sync_copy(rows_t, acc_sh.at[dst_v.at[pl.ds(TOFF, CKT)]],
                        add=True)
        plsc.subcore_barrier()

        pltpu.sync_copy(acc_sh.at[pl.ds(sid * RPT, RPT)],
                        out_hbm.at[pl.ds(sid * RPT, RPT),
                                   pl.ds(cid * width, width)])

        @pl.when(sid == 0)
        def _():
            pltpu.sync_copy(acc_sh.at[pl.ds(NS * RPT, TAIL)],
                            out_hbm.at[pl.ds(NS * RPT, TAIL),
                                       pl.ds(cid * width, width)])

    return pl.kernel(
        body,
        out_type=jax.ShapeDtypeStruct((N, 2 * width), jnp.float32),
        mesh=_MESH,
        scratch_types=[
            pltpu.VMEM((EPW,), jnp.int32),
            pltpu.VMEM((EPW,), jnp.int32),
            pltpu.VMEM((CK, width), jnp.float32),
            pltpu.VMEM((CK, width), jnp.float32),
            pltpu.VMEM((CKT, width), jnp.float32),
            pltpu.VMEM_SHARED((N, width), jnp.float32),
            pltpu.SemaphoreType.DMA,
            pltpu.SemaphoreType.DMA,
            pltpu.SemaphoreType.DMA,
        ],
        compiler_params=_SC_PARAMS,
    )


def _deg_kernel():
    def body(ei_hbm, zeros_hbm, out_hbm, dst_v, ones_v, acc_sh,
             s0, s1, s2, s3):
        ssem = (s0, s1, s2, s3)
        cid = lax.axis_index("c")
        sid = lax.axis_index("s")
        wid = sid * NC + cid

        @pl.when(sid == 0)
        def _():
            pltpu.sync_copy(zeros_hbm, acc_sh.at[pl.ds(0, N)])
        for j in range(CK // 16):
            ones_v[pl.ds(j * 16, 16)] = jnp.full((16,), 1.0, jnp.float32)
        plsc.subcore_barrier()

        pltpu.sync_copy(ei_hbm.at[1].at[pl.ds(wid * EPW, EPW)], dst_v)

        def quad(i, carry):
            base = NBUF * i
            for b in range(NBUF):
                @pl.when(i > 0)
                def _(b=b):
                    pltpu.make_async_copy(
                        ones_v, acc_sh.at[dst_v.at[pl.ds((base - NBUF + b) * CK, CK)]],
                        ssem[b]).wait()
                pltpu.async_copy(ones_v, acc_sh.at[dst_v.at[pl.ds((base + b) * CK, CK)]],
                                 ssem[b], add=True)
            return carry

        lax.fori_loop(0, NQUAD, quad, 0)
        pltpu.make_async_copy(ones_v, acc_sh.at[dst_v.at[pl.ds(0, CK)]],
                              ssem[0]).wait()
        pltpu.async_copy(ones_v,
                         acc_sh.at[dst_v.at[pl.ds((NFULL - 1) * CK, CK)]],
                         ssem[0], add=True)
        pltpu.make_async_copy(ones_v, acc_sh.at[dst_v.at[pl.ds(0, CK)]],
                              ssem[1]).wait()
        pltpu.async_copy(ones_v.at[pl.ds(0, CKT)],
                         acc_sh.at[dst_v.at[pl.ds(TOFF, CKT)]],
                         ssem[1], add=True)
        pltpu.make_async_copy(ones_v, acc_sh.at[dst_v.at[pl.ds(0, CK)]],
                              ssem[0]).wait()
        pltpu.make_async_copy(ones_v.at[pl.ds(0, CKT)],
                              acc_sh.at[dst_v.at[pl.ds(TOFF, CKT)]],
                              ssem[1]).wait()
        for b in range(2, NBUF):
            pltpu.make_async_copy(ones_v, acc_sh.at[dst_v.at[pl.ds(0, CK)]],
                                  ssem[b]).wait()
        plsc.subcore_barrier()

        @pl.when(sid == 0)
        def _():
            pltpu.sync_copy(acc_sh.at[pl.ds(0, N)], out_hbm.at[cid])

    return pl.kernel(
        body,
        out_type=jax.ShapeDtypeStruct((NC, N), jnp.float32),
        mesh=_MESH,
        scratch_types=[
            pltpu.VMEM((EPW,), jnp.int32),
            pltpu.VMEM((CK,), jnp.float32),
            pltpu.VMEM_SHARED((N,), jnp.float32),
        ] + [pltpu.SemaphoreType.DMA] * NBUF,
        compiler_params=_SC_PARAMS,
    )


BN = 2000


def _tc_stage1(x, W1, d0, d1):
    def body(x_ref, w_ref, d0_ref, d1_ref, g1_ref, p1_ref):
        deg = 1.0 + d0_ref[...] + d1_ref[...]
        isr = lax.rsqrt(deg)
        idg = 1.0 / deg
        h = jnp.dot(x_ref[...], w_ref[...], preferred_element_type=jnp.float32)
        g1_ref[...] = h * isr
        p1_ref[:, :H] = h * idg
        p1_ref[:, H:H + 1] = isr
        p1_ref[:, H + 1:H + 2] = idg

    return pl.pallas_call(
        body,
        grid=(N // BN,),
        in_specs=[
            pl.BlockSpec((BN, D), lambda i: (i, 0)),
            pl.BlockSpec((D, H), lambda i: (0, 0)),
            pl.BlockSpec((BN, 1), lambda i: (i, 0)),
            pl.BlockSpec((BN, 1), lambda i: (i, 0)),
        ],
        out_specs=[
            pl.BlockSpec((BN, H), lambda i: (i, 0)),
            pl.BlockSpec((BN, 128), lambda i: (i, 0)),
        ],
        out_shape=[
            jax.ShapeDtypeStruct((N, H), jnp.float32),
            jax.ShapeDtypeStruct((N, 128), jnp.float32),
        ],
    )(x, W1, d0, d1)


def _tc_stage2(agg, p1, b1, W2):
    def body(agg_ref, p1_ref, b1_ref, w_ref, g2_ref, p2_ref):
        a = agg_ref[:, :H] + agg_ref[:, H:]
        isr = p1_ref[:, H:H + 1]
        idg = p1_ref[:, H + 1:H + 2]
        r = jnp.maximum(isr * a + p1_ref[:, :H] + b1_ref[...], 0.0)
        h2 = jnp.dot(r, w_ref[...], preferred_element_type=jnp.float32)
        g2_ref[...] = h2 * isr
        p2_ref[:, :C] = h2 * idg
        p2_ref[:, C:C + 1] = isr

    return pl.pallas_call(
        body,
        grid=(N // BN,),
        in_specs=[
            pl.BlockSpec((BN, 2 * H), lambda i: (i, 0)),
            pl.BlockSpec((BN, 128), lambda i: (i, 0)),
            pl.BlockSpec((1, H), lambda i: (0, 0)),
            pl.BlockSpec((H, C), lambda i: (0, 0)),
        ],
        out_specs=[
            pl.BlockSpec((BN, C), lambda i: (i, 0)),
            pl.BlockSpec((BN, 128), lambda i: (i, 0)),
        ],
        out_shape=[
            jax.ShapeDtypeStruct((N, C), jnp.float32),
            jax.ShapeDtypeStruct((N, 128), jnp.float32),
        ],
    )(agg, p1, b1, W2)


def _tc_stage3(agg, p2, b2):
    def body(agg_ref, p2_ref, b2_ref, out_ref):
        a = agg_ref[:, :C] + agg_ref[:, C:]
        isr = p2_ref[:, C:C + 1]
        out_ref[...] = isr * a + p2_ref[:, :C] + b2_ref[...]

    return pl.pallas_call(
        body,
        grid=(N // BN,),
        in_specs=[
            pl.BlockSpec((BN, 2 * C), lambda i: (i, 0)),
            pl.BlockSpec((BN, 128), lambda i: (i, 0)),
            pl.BlockSpec((1, C), lambda i: (0, 0)),
        ],
        out_specs=pl.BlockSpec((BN, C), lambda i: (i, 0)),
        out_shape=jax.ShapeDtypeStruct((N, C), jnp.float32),
    )(agg, p2, b2)


def kernel(x, edge_index, W1, b1, W2, b2):
    zeros_n = jnp.zeros((N,), jnp.float32)
    zeros_h = jnp.zeros((N, H), jnp.float32)
    zeros_c = jnp.zeros((N, C), jnp.float32)

    ei = edge_index.astype(jnp.int32)
    degp = _deg_kernel()(ei, zeros_n)
    d0 = degp[0].reshape(N, 1)
    d1 = degp[1].reshape(N, 1)

    g1, p1 = _tc_stage1(x, W1, d0, d1)

    agg1 = _seg_sum_kernel(H)(g1, ei, zeros_h)
    g2, p2 = _tc_stage2(agg1, p1, b1.reshape(1, H), W2)

    agg2 = _seg_sum_kernel(C)(g2, ei, zeros_c)
    out = _tc_stage3(agg2, p2, b2.reshape(1, C))
    return out


# speedup vs baseline: 2.1611x; 1.0004x over previous
import jax
import jax.numpy as jnp
from jax import lax
from jax.experimental import pallas as pl
from jax.experimental.pallas import tpu as pltpu
from jax.experimental.pallas import tpu_sc as plsc

N = 10000
E = 320000
D = 128
H = 64
C = 16

NC = 2
NS = 16
NW = NC * NS
EPW = E // NW
CK = 112
NFULL = EPW // CK
CKT = EPW - NFULL * CK
TOFF = NFULL * CK
NBUF = 4
NQUAD = NFULL // NBUF
RPT = 624
TAIL = N - NS * RPT

_MESH = plsc.VectorSubcoreMesh(core_axis_name="c", subcore_axis_name="s")
_SC_PARAMS = pltpu.CompilerParams(use_tc_tiling_on_sc=False)


def _seg_sum_kernel(width):
    def body(g_hbm, ei_hbm, zeros_hbm, out_hbm,
             src_v, dst_v, rows_a, rows_b, rows_t, acc_sh,
             sem_a, sem_b, sem_t):
        cid = lax.axis_index("c")
        sid = lax.axis_index("s")
        wid = sid * NC + cid

        pltpu.sync_copy(zeros_hbm.at[pl.ds(sid * RPT, RPT)],
                        acc_sh.at[pl.ds(sid * RPT, RPT)])

        @pl.when(sid == 0)
        def _():
            pltpu.sync_copy(zeros_hbm.at[pl.ds(NS * RPT, TAIL)],
                            acc_sh.at[pl.ds(NS * RPT, TAIL)])

        plsc.subcore_barrier()

        pltpu.sync_copy(ei_hbm.at[0].at[pl.ds(wid * EPW, EPW)], src_v)
        pltpu.sync_copy(ei_hbm.at[1].at[pl.ds(wid * EPW, EPW)], dst_v)

        def gather(c, buf, sem):
            return pltpu.async_copy(g_hbm.at[src_v.at[pl.ds(c * CK, CK)]],
                                    buf, sem)

        npair = NFULL // 2
        gather(0, rows_a, sem_a)
        gather(1, rows_b, sem_b)
        pltpu.async_copy(g_hbm.at[src_v.at[pl.ds(TOFF, CKT)]], rows_t, sem_t)

        def pair(i, carry):
            c0 = 2 * i
            pltpu.make_async_copy(g_hbm.at[src_v.at[pl.ds(c0 * CK, CK)]],
                                  rows_a, sem_a).wait()
            pltpu.sync_copy(rows_a, acc_sh.at[dst_v.at[pl.ds(c0 * CK, CK)]],
                            add=True)

            @pl.when(c0 + 2 < NFULL)
            def _():
                gather(c0 + 2, rows_a, sem_a)

            pltpu.make_async_copy(g_hbm.at[src_v.at[pl.ds((c0 + 1) * CK, CK)]],
                                  rows_b, sem_b).wait()
            pltpu.sync_copy(rows_b, acc_sh.at[dst_v.at[pl.ds((c0 + 1) * CK, CK)]],
                            add=True)

            @pl.when(c0 + 3 < NFULL)
            def _():
                gather(c0 + 3, rows_b, sem_b)

            return carry

        lax.fori_loop(0, npair, pair, 0)
        if NFULL % 2:
            pltpu.make_async_copy(g_hbm.at[src_v.at[pl.ds((NFULL - 1) * CK, CK)]],
                                  rows_a, sem_a).wait()
            pltpu.sync_copy(rows_a,
                            acc_sh.at[dst_v.at[pl.ds((NFULL - 1) * CK, CK)]],
                            add=True)
        pltpu.make_async_copy(g_hbm.at[src_v.at[pl.ds(TOFF, CKT)]],
                              rows_t, sem_t).wait()
        pltpu.sync_copy(rows_t, acc_sh.at[dst_v.at[pl.ds(TOFF, CKT)]],
                        add=True)
        plsc.subcore_barrier()

        pltpu.sync_copy(acc_sh.at[pl.ds(sid * RPT, RPT)],
                        out_hbm.at[pl.ds(sid * RPT, RPT),
                                   pl.ds(cid * width, width)])

        @pl.when(sid == 0)
        def _():
            pltpu.sync_copy(acc_sh.at[pl.ds(NS * RPT, TAIL)],
                            out_hbm.at[pl.ds(NS * RPT, TAIL),
                                       pl.ds(cid * width, width)])

    return pl.kernel(
        body,
        out_type=jax.ShapeDtypeStruct((N, 2 * width), jnp.float32),
        mesh=_MESH,
        scratch_types=[
            pltpu.VMEM((EPW,), jnp.int32),
            pltpu.VMEM((EPW,), jnp.int32),
            pltpu.VMEM((CK, width), jnp.float32),
            pltpu.VMEM((CK, width), jnp.float32),
            pltpu.VMEM((CKT, width), jnp.float32),
            pltpu.VMEM_SHARED((N, width), jnp.float32),
            pltpu.SemaphoreType.DMA,
            pltpu.SemaphoreType.DMA,
            pltpu.SemaphoreType.DMA,
        ],
        compiler_params=_SC_PARAMS,
    )


def _deg_kernel():
    def body(ei_hbm, zeros_hbm, out_hbm, dst_v, ones_v, acc_sh,
             s0, s1, s2, s3):
        ssem = (s0, s1, s2, s3)
        cid = lax.axis_index("c")
        sid = lax.axis_index("s")
        wid = sid * NC + cid

        @pl.when(sid == 0)
        def _():
            pltpu.sync_copy(zeros_hbm, acc_sh.at[pl.ds(0, N)])
        for j in range(CK // 16):
            ones_v[pl.ds(j * 16, 16)] = jnp.full((16,), 1.0, jnp.float32)
        plsc.subcore_barrier()

        pltpu.sync_copy(ei_hbm.at[1].at[pl.ds(wid * EPW, EPW)], dst_v)

        def quad(i, carry):
            base = NBUF * i
            for b in range(NBUF):
                @pl.when(i > 0)
                def _(b=b):
                    pltpu.make_async_copy(
                        ones_v, acc_sh.at[dst_v.at[pl.ds((base - NBUF + b) * CK, CK)]],
                        ssem[b]).wait()
                pltpu.async_copy(ones_v, acc_sh.at[dst_v.at[pl.ds((base + b) * CK, CK)]],
                                 ssem[b], add=True)
            return carry

        lax.fori_loop(0, NQUAD, quad, 0)

        def wait_full(b):
            pltpu.make_async_copy(ones_v, acc_sh.at[dst_v.at[pl.ds(0, CK)]],
                                  ssem[b]).wait()

        outstanding = {b: (CK if NQUAD > 0 else None) for b in range(NBUF)}
        for r in range(NBUF * NQUAD, NFULL):
            b = r % NBUF
            if outstanding[b]:
                wait_full(b)
            pltpu.async_copy(ones_v, acc_sh.at[dst_v.at[pl.ds(r * CK, CK)]],
                             ssem[b], add=True)
            outstanding[b] = CK
        bt = NFULL % NBUF
        if outstanding[bt]:
            wait_full(bt)
        pltpu.async_copy(ones_v.at[pl.ds(0, CKT)],
                         acc_sh.at[dst_v.at[pl.ds(TOFF, CKT)]],
                         ssem[bt], add=True)
        outstanding[bt] = CKT
        for b in range(NBUF):
            if outstanding[b] == CK:
                wait_full(b)
            elif outstanding[b] == CKT:
                pltpu.make_async_copy(ones_v.at[pl.ds(0, CKT)],
                                      acc_sh.at[dst_v.at[pl.ds(TOFF, CKT)]],
                                      ssem[b]).wait()
        plsc.subcore_barrier()

        @pl.when(sid == 0)
        def _():
            pltpu.sync_copy(acc_sh.at[pl.ds(0, N)], out_hbm.at[cid])

    return pl.kernel(
        body,
        out_type=jax.ShapeDtypeStruct((NC, N), jnp.float32),
        mesh=_MESH,
        scratch_types=[
            pltpu.VMEM((EPW,), jnp.int32),
            pltpu.VMEM((CK,), jnp.float32),
            pltpu.VMEM_SHARED((N,), jnp.float32),
        ] + [pltpu.SemaphoreType.DMA] * NBUF,
        compiler_params=_SC_PARAMS,
    )


BN = 2000


def _tc_stage1(x, W1, d0, d1):
    def body(x_ref, w_ref, d0_ref, d1_ref, g1_ref, p1_ref):
        deg = 1.0 + d0_ref[...] + d1_ref[...]
        isr = lax.rsqrt(deg)
        idg = 1.0 / deg
        h = jnp.dot(x_ref[...], w_ref[...], preferred_element_type=jnp.float32)
        g1_ref[...] = h * isr
        p1_ref[:, :H] = h * idg
        p1_ref[:, H:H + 1] = isr
        p1_ref[:, H + 1:H + 2] = idg

    return pl.pallas_call(
        body,
        grid=(N // BN,),
        in_specs=[
            pl.BlockSpec((BN, D), lambda i: (i, 0)),
            pl.BlockSpec((D, H), lambda i: (0, 0)),
            pl.BlockSpec((BN, 1), lambda i: (i, 0)),
            pl.BlockSpec((BN, 1), lambda i: (i, 0)),
        ],
        out_specs=[
            pl.BlockSpec((BN, H), lambda i: (i, 0)),
            pl.BlockSpec((BN, 128), lambda i: (i, 0)),
        ],
        out_shape=[
            jax.ShapeDtypeStruct((N, H), jnp.float32),
            jax.ShapeDtypeStruct((N, 128), jnp.float32),
        ],
    )(x, W1, d0, d1)


def _tc_stage2(agg, p1, b1, W2):
    def body(agg_ref, p1_ref, b1_ref, w_ref, g2_ref, p2_ref):
        a = agg_ref[:, :H] + agg_ref[:, H:]
        isr = p1_ref[:, H:H + 1]
        idg = p1_ref[:, H + 1:H + 2]
        r = jnp.maximum(isr * a + p1_ref[:, :H] + b1_ref[...], 0.0)
        h2 = jnp.dot(r, w_ref[...], preferred_element_type=jnp.float32)
        g2_ref[...] = h2 * isr
        p2_ref[:, :C] = h2 * idg
        p2_ref[:, C:C + 1] = isr

    return pl.pallas_call(
        body,
        grid=(N // BN,),
        in_specs=[
            pl.BlockSpec((BN, 2 * H), lambda i: (i, 0)),
            pl.BlockSpec((BN, 128), lambda i: (i, 0)),
            pl.BlockSpec((1, H), lambda i: (0, 0)),
            pl.BlockSpec((H, C), lambda i: (0, 0)),
        ],
        out_specs=[
            pl.BlockSpec((BN, C), lambda i: (i, 0)),
            pl.BlockSpec((BN, 128), lambda i: (i, 0)),
        ],
        out_shape=[
            jax.ShapeDtypeStruct((N, C), jnp.float32),
            jax.ShapeDtypeStruct((N, 128), jnp.float32),
        ],
    )(agg, p1, b1, W2)


def _tc_stage3(agg, p2, b2):
    def body(agg_ref, p2_ref, b2_ref, out_ref):
        a = agg_ref[:, :C] + agg_ref[:, C:]
        isr = p2_ref[:, C:C + 1]
        out_ref[...] = isr * a + p2_ref[:, :C] + b2_ref[...]

    return pl.pallas_call(
        body,
        grid=(N // BN,),
        in_specs=[
            pl.BlockSpec((BN, 2 * C), lambda i: (i, 0)),
            pl.BlockSpec((BN, 128), lambda i: (i, 0)),
            pl.BlockSpec((1, C), lambda i: (0, 0)),
        ],
        out_specs=pl.BlockSpec((BN, C), lambda i: (i, 0)),
        out_shape=jax.ShapeDtypeStruct((N, C), jnp.float32),
    )(agg, p2, b2)


def kernel(x, edge_index, W1, b1, W2, b2):
    zeros_n = jnp.zeros((N,), jnp.float32)
    zeros_h = jnp.zeros((N, H), jnp.float32)
    zeros_c = jnp.zeros((N, C), jnp.float32)

    ei = edge_index.astype(jnp.int32)
    degp = _deg_kernel()(ei, zeros_n)
    d0 = degp[0].reshape(N, 1)
    d1 = degp[1].reshape(N, 1)

    g1, p1 = _tc_stage1(x, W1, d0, d1)

    agg1 = _seg_sum_kernel(H)(g1, ei, zeros_h)
    g2, p2 = _tc_stage2(agg1, p1, b1.reshape(1, H), W2)

    agg2 = _seg_sum_kernel(C)(g2, ei, zeros_c)
    out = _tc_stage3(agg2, p2, b2.reshape(1, C))
    return out


# speedup vs baseline: 2.2326x; 1.0331x over previous
import jax
import jax.numpy as jnp
from jax import lax
from jax.experimental import pallas as pl
from jax.experimental.pallas import tpu as pltpu
from jax.experimental.pallas import tpu_sc as plsc

N = 10000
E = 320000
D = 128
H = 64
C = 16

NC = 2
NS = 16
NW = NC * NS
EPW = E // NW
CK = 128
NFULL = EPW // CK
CKT = EPW - NFULL * CK
TOFF = NFULL * CK
NBUF = 4
NQUAD = NFULL // NBUF
RPT = 624
TAIL = N - NS * RPT

_MESH = plsc.VectorSubcoreMesh(core_axis_name="c", subcore_axis_name="s")
_SC_PARAMS = pltpu.CompilerParams(use_tc_tiling_on_sc=False)


def _seg_sum_kernel(width):
    def body(g_hbm, ei_hbm, zeros_hbm, out_hbm,
             src_v, dst_v, rows_a, rows_b, rows_t, acc_sh,
             sem_a, sem_b, sem_t):
        cid = lax.axis_index("c")
        sid = lax.axis_index("s")
        wid = sid * NC + cid

        pltpu.sync_copy(zeros_hbm.at[pl.ds(sid * RPT, RPT)],
                        acc_sh.at[pl.ds(sid * RPT, RPT)])

        @pl.when(sid == 0)
        def _():
            pltpu.sync_copy(zeros_hbm.at[pl.ds(NS * RPT, TAIL)],
                            acc_sh.at[pl.ds(NS * RPT, TAIL)])

        plsc.subcore_barrier()

        pltpu.sync_copy(ei_hbm.at[0].at[pl.ds(wid * EPW, EPW)], src_v)
        pltpu.sync_copy(ei_hbm.at[1].at[pl.ds(wid * EPW, EPW)], dst_v)

        def gather(c, buf, sem):
            return pltpu.async_copy(g_hbm.at[src_v.at[pl.ds(c * CK, CK)]],
                                    buf, sem)

        npair = NFULL // 2
        gather(0, rows_a, sem_a)
        gather(1, rows_b, sem_b)
        pltpu.async_copy(g_hbm.at[src_v.at[pl.ds(TOFF, CKT)]], rows_t, sem_t)

        def pair(i, carry):
            c0 = 2 * i
            pltpu.make_async_copy(g_hbm.at[src_v.at[pl.ds(c0 * CK, CK)]],
                                  rows_a, sem_a).wait()
            pltpu.sync_copy(rows_a, acc_sh.at[dst_v.at[pl.ds(c0 * CK, CK)]],
                            add=True)

            @pl.when(c0 + 2 < NFULL)
            def _():
                gather(c0 + 2, rows_a, sem_a)

            pltpu.make_async_copy(g_hbm.at[src_v.at[pl.ds((c0 + 1) * CK, CK)]],
                                  rows_b, sem_b).wait()
            pltpu.sync_copy(rows_b, acc_sh.at[dst_v.at[pl.ds((c0 + 1) * CK, CK)]],
                            add=True)

            @pl.when(c0 + 3 < NFULL)
            def _():
                gather(c0 + 3, rows_b, sem_b)

            return carry

        lax.fori_loop(0, npair, pair, 0)
        if NFULL % 2:
            pltpu.make_async_copy(g_hbm.at[src_v.at[pl.ds((NFULL - 1) * CK, CK)]],
                                  rows_a, sem_a).wait()
            pltpu.sync_copy(rows_a,
                            acc_sh.at[dst_v.at[pl.ds((NFULL - 1) * CK, CK)]],
                            add=True)
        pltpu.make_async_copy(g_hbm.at[src_v.at[pl.ds(TOFF, CKT)]],
                              rows_t, sem_t).wait()
        pltpu.sync_copy(rows_t, acc_sh.at[dst_v.at[pl.ds(TOFF, CKT)]],
                        add=True)
        plsc.subcore_barrier()

        pltpu.sync_copy(acc_sh.at[pl.ds(sid * RPT, RPT)],
                        out_hbm.at[pl.ds(sid * RPT, RPT),
                                   pl.ds(cid * width, width)])

        @pl.when(sid == 0)
        def _():
            pltpu.sync_copy(acc_sh.at[pl.ds(NS * RPT, TAIL)],
                            out_hbm.at[pl.ds(NS * RPT, TAIL),
                                       pl.ds(cid * width, width)])

    return pl.kernel(
        body,
        out_type=jax.ShapeDtypeStruct((N, 2 * width), jnp.float32),
        mesh=_MESH,
        scratch_types=[
            pltpu.VMEM((EPW,), jnp.int32),
            pltpu.VMEM((EPW,), jnp.int32),
            pltpu.VMEM((CK, width), jnp.float32),
            pltpu.VMEM((CK, width), jnp.float32),
            pltpu.VMEM((CKT, width), jnp.float32),
            pltpu.VMEM_SHARED((N, width), jnp.float32),
            pltpu.SemaphoreType.DMA,
            pltpu.SemaphoreType.DMA,
            pltpu.SemaphoreType.DMA,
        ],
        compiler_params=_SC_PARAMS,
    )


def _deg_kernel():
    def body(ei_hbm, zeros_hbm, out_hbm, dst_v, ones_v, acc_sh,
             s0, s1, s2, s3):
        ssem = (s0, s1, s2, s3)
        cid = lax.axis_index("c")
        sid = lax.axis_index("s")
        wid = sid * NC + cid

        @pl.when(sid == 0)
        def _():
            pltpu.sync_copy(zeros_hbm, acc_sh.at[pl.ds(0, N)])
        for j in range(CK // 16):
            ones_v[pl.ds(j * 16, 16)] = jnp.full((16,), 1.0, jnp.float32)
        plsc.subcore_barrier()

        pltpu.sync_copy(ei_hbm.at[1].at[pl.ds(wid * EPW, EPW)], dst_v)

        def quad(i, carry):
            base = NBUF * i
            for b in range(NBUF):
                @pl.when(i > 0)
                def _(b=b):
                    pltpu.make_async_copy(
                        ones_v, acc_sh.at[dst_v.at[pl.ds((base - NBUF + b) * CK, CK)]],
                        ssem[b]).wait()
                pltpu.async_copy(ones_v, acc_sh.at[dst_v.at[pl.ds((base + b) * CK, CK)]],
                                 ssem[b], add=True)
            return carry

        lax.fori_loop(0, NQUAD, quad, 0)

        def wait_full(b):
            pltpu.make_async_copy(ones_v, acc_sh.at[dst_v.at[pl.ds(0, CK)]],
                                  ssem[b]).wait()

        outstanding = {b: (CK if NQUAD > 0 else None) for b in range(NBUF)}
        for r in range(NBUF * NQUAD, NFULL):
            b = r % NBUF
            if outstanding[b]:
                wait_full(b)
            pltpu.async_copy(ones_v, acc_sh.at[dst_v.at[pl.ds(r * CK, CK)]],
                             ssem[b], add=True)
            outstanding[b] = CK
        bt = NFULL % NBUF
        if outstanding[bt]:
            wait_full(bt)
        pltpu.async_copy(ones_v.at[pl.ds(0, CKT)],
                         acc_sh.at[dst_v.at[pl.ds(TOFF, CKT)]],
                         ssem[bt], add=True)
        outstanding[bt] = CKT
        for b in range(NBUF):
            if outstanding[b] == CK:
                wait_full(b)
            elif outstanding[b] == CKT:
                pltpu.make_async_copy(ones_v.at[pl.ds(0, CKT)],
                                      acc_sh.at[dst_v.at[pl.ds(TOFF, CKT)]],
                                      ssem[b]).wait()
        plsc.subcore_barrier()

        @pl.when(sid == 0)
        def _():
            pltpu.sync_copy(acc_sh.at[pl.ds(0, N)], out_hbm.at[cid])

    return pl.kernel(
        body,
        out_type=jax.ShapeDtypeStruct((NC, N), jnp.float32),
        mesh=_MESH,
        scratch_types=[
            pltpu.VMEM((EPW,), jnp.int32),
            pltpu.VMEM((CK,), jnp.float32),
            pltpu.VMEM_SHARED((N,), jnp.float32),
        ] + [pltpu.SemaphoreType.DMA] * NBUF,
        compiler_params=_SC_PARAMS,
    )


BN = 2000


def _tc_stage1(x, W1, d0, d1):
    def body(x_ref, w_ref, d0_ref, d1_ref, g1_ref, p1_ref):
        deg = 1.0 + d0_ref[...] + d1_ref[...]
        isr = lax.rsqrt(deg)
        idg = 1.0 / deg
        h = jnp.dot(x_ref[...], w_ref[...], preferred_element_type=jnp.float32)
        g1_ref[...] = h * isr
        p1_ref[:, :H] = h * idg
        p1_ref[:, H:H + 1] = isr
        p1_ref[:, H + 1:H + 2] = idg

    return pl.pallas_call(
        body,
        grid=(N // BN,),
        in_specs=[
            pl.BlockSpec((BN, D), lambda i: (i, 0)),
            pl.BlockSpec((D, H), lambda i: (0, 0)),
            pl.BlockSpec((BN, 1), lambda i: (i, 0)),
            pl.BlockSpec((BN, 1), lambda i: (i, 0)),
        ],
        out_specs=[
            pl.BlockSpec((BN, H), lambda i: (i, 0)),
            pl.BlockSpec((BN, 128), lambda i: (i, 0)),
        ],
        out_shape=[
            jax.ShapeDtypeStruct((N, H), jnp.float32),
            jax.ShapeDtypeStruct((N, 128), jnp.float32),
        ],
    )(x, W1, d0, d1)


def _tc_stage2(agg, p1, b1, W2):
    def body(agg_ref, p1_ref, b1_ref, w_ref, g2_ref, p2_ref):
        a = agg_ref[:, :H] + agg_ref[:, H:]
        isr = p1_ref[:, H:H + 1]
        idg = p1_ref[:, H + 1:H + 2]
        r = jnp.maximum(isr * a + p1_ref[:, :H] + b1_ref[...], 0.0)
        h2 = jnp.dot(r, w_ref[...], preferred_element_type=jnp.float32)
        g2_ref[...] = h2 * isr
        p2_ref[:, :C] = h2 * idg
        p2_ref[:, C:C + 1] = isr

    return pl.pallas_call(
        body,
        grid=(N // BN,),
        in_specs=[
            pl.BlockSpec((BN, 2 * H), lambda i: (i, 0)),
            pl.BlockSpec((BN, 128), lambda i: (i, 0)),
            pl.BlockSpec((1, H), lambda i: (0, 0)),
            pl.BlockSpec((H, C), lambda i: (0, 0)),
        ],
        out_specs=[
            pl.BlockSpec((BN, C), lambda i: (i, 0)),
            pl.BlockSpec((BN, 128), lambda i: (i, 0)),
        ],
        out_shape=[
            jax.ShapeDtypeStruct((N, C), jnp.float32),
            jax.ShapeDtypeStruct((N, 128), jnp.float32),
        ],
    )(agg, p1, b1, W2)


def _tc_stage3(agg, p2, b2):
    def body(agg_ref, p2_ref, b2_ref, out_ref):
        a = agg_ref[:, :C] + agg_ref[:, C:]
        isr = p2_ref[:, C:C + 1]
        out_ref[...] = isr * a + p2_ref[:, :C] + b2_ref[...]

    return pl.pallas_call(
        body,
        grid=(N // BN,),
        in_specs=[
            pl.BlockSpec((BN, 2 * C), lambda i: (i, 0)),
            pl.BlockSpec((BN, 128), lambda i: (i, 0)),
            pl.BlockSpec((1, C), lambda i: (0, 0)),
        ],
        out_specs=pl.BlockSpec((BN, C), lambda i: (i, 0)),
        out_shape=jax.ShapeDtypeStruct((N, C), jnp.float32),
    )(agg, p2, b2)


def kernel(x, edge_index, W1, b1, W2, b2):
    zeros_n = jnp.zeros((N,), jnp.float32)
    zeros_h = jnp.zeros((N, H), jnp.float32)
    zeros_c = jnp.zeros((N, C), jnp.float32)

    ei = edge_index.astype(jnp.int32)
    degp = _deg_kernel()(ei, zeros_n)
    d0 = degp[0].reshape(N, 1)
    d1 = degp[1].reshape(N, 1)

    g1, p1 = _tc_stage1(x, W1, d0, d1)

    agg1 = _seg_sum_kernel(H)(g1, ei, zeros_h)
    g2, p2 = _tc_stage2(agg1, p1, b1.reshape(1, H), W2)

    agg2 = _seg_sum_kernel(C)(g2, ei, zeros_c)
    out = _tc_stage3(agg2, p2, b2.reshape(1, C))
    return out
